# baseline (speedup 1.0000x reference)
import functools

import jax
import jax.numpy as jnp
from jax import lax
from jax.experimental import pallas as pl
from jax.experimental.pallas import tpu as pltpu
from jax.experimental.pallas import tpu_sc as plsc

_D = 32
_B = 16384
_NC = 2
_NS = 16
_NW = _NC * _NS
_BPW = _B // _NW
_CHUNK = 128
_NCHUNK = _BPW // _CHUNK
_GROUP = 16
_NGRP_C = _CHUNK // _GROUP
_ROWS_PER_LINE = 4
_NLINES = 1000000 // _ROWS_PER_LINE

_mesh = plsc.VectorSubcoreMesh(core_axis_name="c", subcore_axis_name="s")


@functools.partial(
    pl.kernel,
    mesh=_mesh,
    out_type=jax.ShapeDtypeStruct((_B,), jnp.float32),
    compiler_params=pltpu.CompilerParams(needs_layout_passes=False),
    scratch_types=[
        pltpu.VMEM((_BPW,), jnp.int32),
        pltpu.VMEM((_BPW,), jnp.int32),
        pltpu.VMEM((_NCHUNK, _CHUNK), jnp.int32),
        pltpu.VMEM((_NCHUNK, _CHUNK), jnp.int32),
        pltpu.VMEM((2, _CHUNK, 128), jnp.float32),
        pltpu.VMEM((2, _CHUNK, 128), jnp.float32),
        pltpu.VMEM((_D,), jnp.float32),
        pltpu.VMEM((16,), jnp.float32),
        pltpu.VMEM((_GROUP * 16,), jnp.float32),
        pltpu.VMEM((_BPW,), jnp.float32),
        pltpu.SemaphoreType.DMA,
        pltpu.SemaphoreType.DMA,
    ],
)
def _gmf_sc(uidx_hbm, iidx_hbm, utab_hbm, itab_hbm, w_hbm, b_hbm, out_hbm,
            uidx_v, iidx_v, uline_v, iline_v, urows_v, irows_v, w_v, b_v,
            sred_v, out_v, sem0, sem1):
    cid = lax.axis_index("c")
    sid = lax.axis_index("s")
    wid = sid * _NC + cid
    base = wid * _BPW
    sems = [sem0, sem1]

    pltpu.sync_copy(uidx_hbm.at[pl.ds(base, _BPW)], uidx_v)
    pltpu.sync_copy(iidx_hbm.at[pl.ds(base, _BPW)], iidx_v)
    pltpu.sync_copy(w_hbm, w_v)
    pltpu.sync_copy(b_hbm, b_v)

    def line_ids(g, carry):
        ju = uidx_v[pl.ds(g * _GROUP, _GROUP)]
        ji = iidx_v[pl.ds(g * _GROUP, _GROUP)]
        c = g // _NGRP_C
        o = (g % _NGRP_C) * _GROUP
        uline_v[c, pl.ds(o, _GROUP)] = jax.lax.shift_right_logical(ju, 2)
        iline_v[c, pl.ds(o, _GROUP)] = jax.lax.shift_right_logical(ji, 2)
        return carry

    for g in range(_BPW // _GROUP):
        line_ids(g, 0)

    def fire(c):
        buf = c % 2
        pltpu.async_copy(utab_hbm.at[uline_v.at[c]],
                         urows_v.at[buf], sems[buf])
        pltpu.async_copy(itab_hbm.at[iline_v.at[c]],
                         irows_v.at[buf], sems[buf])

    def drain(c):
        buf = c % 2
        pltpu.make_async_copy(
            utab_hbm.at[pl.ds(0, _CHUNK)], urows_v.at[buf], sems[buf]).wait()
        pltpu.make_async_copy(
            itab_hbm.at[pl.ds(0, _CHUNK)], irows_v.at[buf], sems[buf]).wait()

    bias = b_v[...]
    w_lo = w_v[pl.ds(0, 16)]
    w_hi = w_v[pl.ds(16, 16)]
    lane = lax.iota(jnp.int32, 16)
    col_idx = [lane * 16 + c for c in range(16)]

    fire(0)
    for c in range(_NCHUNK):
        if c + 1 < _NCHUNK:
            fire(c + 1)
        drain(c)
        buf = c % 2
        for g in range(_NGRP_C):
            rbase = g * _GROUP
            ju = uidx_v[pl.ds(c * _CHUNK + rbase, _GROUP)]
            ji = iidx_v[pl.ds(c * _CHUNK + rbase, _GROUP)]
            uoff = (ju & 3) * _D
            ioff = (ji & 3) * _D
            for r in range(_GROUP):
                row = rbase + r
                uo = uoff[r]
                io = ioff[r]
                u0 = urows_v[buf, row, pl.ds(uo, 16)]
                u1 = urows_v[buf, row, pl.ds(uo + 16, 16)]
                i0 = irows_v[buf, row, pl.ds(io, 16)]
                i1 = irows_v[buf, row, pl.ds(io + 16, 16)]
                sred_v[pl.ds(r * 16, 16)] = u0 * i0 * w_lo + u1 * i1 * w_hi
            acc = bias
            for k in range(16):
                acc = acc + plsc.load_gather(sred_v, [col_idx[k]])
            out_v[pl.ds(c * _CHUNK + rbase, _GROUP)] = (
                1.0 / (1.0 + jnp.exp(-acc)))

    pltpu.sync_copy(out_v, out_hbm.at[pl.ds(base, _BPW)])


def kernel(user_indices, item_indices, user_table, item_table, W, b):
    uidx = user_indices.astype(jnp.int32)
    iidx = item_indices.astype(jnp.int32)
    utab = user_table.reshape(_NLINES, 128)
    itab = item_table.reshape(_NLINES, 128)
    wvec = W.reshape(_D)
    bvec = jnp.broadcast_to(b.reshape(1), (16,))
    return _gmf_sc(uidx, iidx, utab, itab, wvec, bvec)


# speedup vs baseline: 3.6805x; 3.6805x over previous
import functools

import jax
import jax.numpy as jnp
from jax import lax
from jax.experimental import pallas as pl
from jax.experimental.pallas import tpu as pltpu
from jax.experimental.pallas import tpu_sc as plsc

_D = 32
_B = 16384
_NC = 2
_NS = 16
_NW = _NC * _NS
_BPW = _B // _NW
_GROUP = 16
_NGROUP = _BPW // _GROUP
_FLIGHT = 4
_NFLIGHT = _GROUP // _FLIGHT

_mesh = plsc.VectorSubcoreMesh(core_axis_name="c", subcore_axis_name="s")


@functools.partial(
    pl.kernel,
    mesh=_mesh,
    out_type=jax.ShapeDtypeStruct((_B,), jnp.float32),
    compiler_params=pltpu.CompilerParams(needs_layout_passes=False),
    scratch_types=[
        pltpu.VMEM((_BPW,), jnp.int32),
        pltpu.VMEM((_BPW,), jnp.int32),
        pltpu.VMEM((_D,), jnp.int32),
        pltpu.VMEM((2 * _FLIGHT * _D, 128), jnp.float32),
        pltpu.VMEM((2 * _FLIGHT * _D, 128), jnp.float32),
        pltpu.VMEM((_D,), jnp.float32),
        pltpu.VMEM((16,), jnp.float32),
        pltpu.VMEM((_GROUP * 16,), jnp.float32),
        pltpu.VMEM((_BPW,), jnp.float32),
        pltpu.SemaphoreType.DMA,
        pltpu.SemaphoreType.DMA,
    ],
)
def _gmf_sc(uidx_hbm, iidx_hbm, utab_hbm, itab_hbm, w_hbm, b_hbm, out_hbm,
            uidx_v, iidx_v, diota_v, uslab_v, islab_v, w_v, b_v, sred_v,
            out_v, sem0, sem1):
    cid = lax.axis_index("c")
    sid = lax.axis_index("s")
    wid = sid * _NC + cid
    base = wid * _BPW
    sems = [sem0, sem1]

    pltpu.sync_copy(uidx_hbm.at[pl.ds(base, _BPW)], uidx_v)
    pltpu.sync_copy(iidx_hbm.at[pl.ds(base, _BPW)], iidx_v)
    pltpu.sync_copy(w_hbm, w_v)
    pltpu.sync_copy(b_hbm, b_v)

    lane = lax.iota(jnp.int32, 16)
    diota_v[pl.ds(0, 16)] = lane
    diota_v[pl.ds(16, 16)] = lane + 16

    bias = b_v[...]
    w_lo = w_v[pl.ds(0, 16)]
    w_hi = w_v[pl.ds(16, 16)]
    col_idx = [lane * 16 + c for c in range(16)]

    def fire(q, ju_vec, ji_vec):
        buf = q % 2
        copies = []
        for r in range(_FLIGHT):
            ju = ju_vec[q * _FLIGHT + r]
            ji = ji_vec[q * _FLIGHT + r]
            cu = pl.multiple_of(jax.lax.shift_right_logical(ju, 7) * 128, 128)
            ci = pl.multiple_of(jax.lax.shift_right_logical(ji, 7) * 128, 128)
            dst = pl.ds((buf * _FLIGHT + r) * _D, _D)
            copies.append(pltpu.async_copy(
                utab_hbm.at[diota_v, pl.ds(cu, 128)],
                uslab_v.at[dst, pl.ds(0, 128)], sems[buf]))
            copies.append(pltpu.async_copy(
                itab_hbm.at[diota_v, pl.ds(ci, 128)],
                islab_v.at[dst, pl.ds(0, 128)], sems[buf]))
        return copies

    def group_body(g, carry):
        gbase = g * _GROUP
        ju_vec = uidx_v[pl.ds(gbase, _GROUP)]
        ji_vec = iidx_v[pl.ds(gbase, _GROUP)]
        inflight = fire(0, ju_vec, ji_vec)
        for q in range(_NFLIGHT):
            nxt = fire(q + 1, ju_vec, ji_vec) if q + 1 < _NFLIGHT else []
            for c in inflight:
                c.wait()
            inflight = nxt
            buf = q % 2
            for r in range(_FLIGHT):
                i = q * _FLIGHT + r
                lu = jnp.full((16,), ju_vec[i] & 127, jnp.int32)
                li = jnp.full((16,), ji_vec[i] & 127, jnp.int32)
                rbase = (buf * _FLIGHT + r) * _D
                u0 = plsc.load_gather(uslab_v, [lane + rbase, lu])
                u1 = plsc.load_gather(uslab_v, [lane + (rbase + 16), lu])
                i0 = plsc.load_gather(islab_v, [lane + rbase, li])
                i1 = plsc.load_gather(islab_v, [lane + (rbase + 16), li])
                sred_v[pl.ds(i * 16, 16)] = u0 * i0 * w_lo + u1 * i1 * w_hi
        acc = bias
        for k in range(16):
            acc = acc + plsc.load_gather(sred_v, [col_idx[k]])
        out_v[pl.ds(gbase, _GROUP)] = 1.0 / (1.0 + jnp.exp(-acc))
        return carry

    lax.fori_loop(0, _NGROUP, group_body, 0)

    pltpu.sync_copy(out_v, out_hbm.at[pl.ds(base, _BPW)])


def kernel(user_indices, item_indices, user_table, item_table, W, b):
    uidx = user_indices.astype(jnp.int32)
    iidx = item_indices.astype(jnp.int32)
    wvec = W.reshape(_D)
    bvec = jnp.broadcast_to(b.reshape(1), (16,))
    return _gmf_sc(uidx, iidx, user_table.T, item_table.T, wvec, bvec)


# speedup vs baseline: 3.6815x; 1.0003x over previous
import functools

import jax
import jax.numpy as jnp
from jax import lax
from jax.experimental import pallas as pl
from jax.experimental.pallas import tpu as pltpu
from jax.experimental.pallas import tpu_sc as plsc

_D = 32
_B = 16384
_NC = 2
_NS = 16
_NW = _NC * _NS
_BPW = _B // _NW
_GROUP = 16
_NGROUP = _BPW // _GROUP
_FLIGHT = 4
_NFLIGHT = _GROUP // _FLIGHT

_mesh = plsc.VectorSubcoreMesh(core_axis_name="c", subcore_axis_name="s")


@functools.partial(
    pl.kernel,
    mesh=_mesh,
    out_type=jax.ShapeDtypeStruct((_B,), jnp.float32),
    compiler_params=pltpu.CompilerParams(needs_layout_passes=False),
    scratch_types=[
        pltpu.VMEM((_BPW,), jnp.int32),
        pltpu.VMEM((_BPW,), jnp.int32),
        pltpu.VMEM((_D,), jnp.int32),
        pltpu.VMEM((3 * _FLIGHT * _D, 128), jnp.float32),
        pltpu.VMEM((3 * _FLIGHT * _D, 128), jnp.float32),
        pltpu.VMEM((_D,), jnp.float32),
        pltpu.VMEM((16,), jnp.float32),
        pltpu.VMEM((_GROUP * 16,), jnp.float32),
        pltpu.VMEM((_BPW,), jnp.float32),
        pltpu.SemaphoreType.DMA,
        pltpu.SemaphoreType.DMA,
        pltpu.SemaphoreType.DMA,
    ],
)
def _gmf_sc(uidx_hbm, iidx_hbm, utab_hbm, itab_hbm, w_hbm, b_hbm, out_hbm,
            uidx_v, iidx_v, diota_v, uslab_v, islab_v, w_v, b_v, sred_v,
            out_v, sem0, sem1, sem2):
    cid = lax.axis_index("c")
    sid = lax.axis_index("s")
    wid = sid * _NC + cid
    base = wid * _BPW
    sems = [sem0, sem1, sem2]

    pltpu.sync_copy(uidx_hbm.at[pl.ds(base, _BPW)], uidx_v)
    pltpu.sync_copy(iidx_hbm.at[pl.ds(base, _BPW)], iidx_v)
    pltpu.sync_copy(w_hbm, w_v)
    pltpu.sync_copy(b_hbm, b_v)

    lane = lax.iota(jnp.int32, 16)
    diota_v[pl.ds(0, 16)] = lane
    diota_v[pl.ds(16, 16)] = lane + 16

    bias = b_v[...]
    w_lo = w_v[pl.ds(0, 16)]
    w_hi = w_v[pl.ds(16, 16)]
    col_idx = [lane * 16 + c for c in range(16)]

    def fire(q, ju_vec, ji_vec):
        buf = q % 3
        copies = []
        for r in range(_FLIGHT):
            ju = ju_vec[q * _FLIGHT + r]
            ji = ji_vec[q * _FLIGHT + r]
            cu = pl.multiple_of(jax.lax.shift_right_logical(ju, 7) * 128, 128)
            ci = pl.multiple_of(jax.lax.shift_right_logical(ji, 7) * 128, 128)
            dst = pl.ds((buf * _FLIGHT + r) * _D, _D)
            copies.append(pltpu.async_copy(
                utab_hbm.at[diota_v, pl.ds(cu, 128)],
                uslab_v.at[dst, pl.ds(0, 128)], sems[buf]))
            copies.append(pltpu.async_copy(
                itab_hbm.at[diota_v, pl.ds(ci, 128)],
                islab_v.at[dst, pl.ds(0, 128)], sems[buf]))
        return copies

    def group_body(g, carry):
        gbase = g * _GROUP
        ju_vec = uidx_v[pl.ds(gbase, _GROUP)]
        ji_vec = iidx_v[pl.ds(gbase, _GROUP)]
        flights = [fire(0, ju_vec, ji_vec), fire(1, ju_vec, ji_vec)]
        for q in range(_NFLIGHT):
            if q + 2 < _NFLIGHT:
                flights.append(fire(q + 2, ju_vec, ji_vec))
            for c in flights.pop(0):
                c.wait()
            buf = q % 3
            for r in range(_FLIGHT):
                i = q * _FLIGHT + r
                lu = jnp.full((16,), ju_vec[i] & 127, jnp.int32)
                li = jnp.full((16,), ji_vec[i] & 127, jnp.int32)
                rbase = (buf * _FLIGHT + r) * _D
                u0 = plsc.load_gather(uslab_v, [lane + rbase, lu])
                u1 = plsc.load_gather(uslab_v, [lane + (rbase + 16), lu])
                i0 = plsc.load_gather(islab_v, [lane + rbase, li])
                i1 = plsc.load_gather(islab_v, [lane + (rbase + 16), li])
                sred_v[pl.ds(i * 16, 16)] = u0 * i0 * w_lo + u1 * i1 * w_hi
        acc = bias
        for k in range(16):
            acc = acc + plsc.load_gather(sred_v, [col_idx[k]])
        out_v[pl.ds(gbase, _GROUP)] = 1.0 / (1.0 + jnp.exp(-acc))
        return carry

    lax.fori_loop(0, _NGROUP, group_body, 0)

    pltpu.sync_copy(out_v, out_hbm.at[pl.ds(base, _BPW)])


def kernel(user_indices, item_indices, user_table, item_table, W, b):
    uidx = user_indices.astype(jnp.int32)
    iidx = item_indices.astype(jnp.int32)
    wvec = W.reshape(_D)
    bvec = jnp.broadcast_to(b.reshape(1), (16,))
    return _gmf_sc(uidx, iidx, user_table.T, item_table.T, wvec, bvec)
